# baseline (speedup 1.0000x reference)
import functools

import jax
import numpy as np

jax.config.update('jax_enable_x64', True)

import jax.numpy as jnp
from jax import lax
from jax.experimental import pallas as pl
from jax.experimental.pallas import tpu as pltpu
from jax.experimental.pallas import tpu_sc as plsc

_B, _D, _N = 4, 3, 4096
_C = 512
_ALPHA = 10.0
_EPS = 1e-6
_BIG = 2**30
_Z = np.int32(0)


def _i32(v):
    return v if v.dtype == jnp.int32 else v.astype(jnp.int32)

_NC, _NS = 2, 16
_PPC = 4
_NP = _B * 2
_CHUNK = _N // (_NS // _PPC)
_ROWS = _CHUNK // 128


_MASK = 0xFFF


def _dist_body(xa_ref, ga_ref, e1_ref, i1_ref, e2_ref, i2_ref, k2_scr):
    ga = ga_ref[0]
    k2_scr[...] = jnp.full((1, _N), jnp.inf, jnp.float32)

    def blk(i, carry):
        xc = xa_ref[0, pl.ds(i * _C, _C), :]
        p = lax.dot_general(xc, ga, (((1,), (0,)), ((), ())),
                            preferred_element_type=jnp.float32,
                            precision=lax.Precision.HIGHEST)
        t = lax.bitcast_convert_type(p, jnp.int32) & jnp.int32(~_MASK)
        lane = lax.broadcasted_iota(jnp.int32, p.shape, 1)
        sub = lax.broadcasted_iota(jnp.int32, p.shape, 0) + (i * _C)
        key1 = lax.bitcast_convert_type(t | lane, jnp.float32)
        key2 = lax.bitcast_convert_type(t | sub, jnp.float32)
        m1 = jnp.min(key1, axis=1)
        b1 = lax.bitcast_convert_type(m1, jnp.int32)
        d1 = lax.bitcast_convert_type(b1 & jnp.int32(~_MASK), jnp.float32)
        e1_ref[0, 0, pl.ds(i * _C, _C)] = jnp.exp(d1 * jnp.float32(-_ALPHA))
        i1_ref[0, 0, pl.ds(i * _C, _C)] = b1 & _MASK
        m2 = jnp.min(key2, axis=0, keepdims=True)
        k2_scr[...] = jnp.minimum(k2_scr[...], m2)
        return carry

    lax.fori_loop(jnp.int32(0), jnp.int32(_N // _C), blk, jnp.int32(0))
    b2 = lax.bitcast_convert_type(k2_scr[0, :], jnp.int32)
    d2 = lax.bitcast_convert_type(b2 & jnp.int32(~_MASK), jnp.float32)
    e2_ref[0, 0, :] = jnp.exp(d2 * jnp.float32(-_ALPHA))
    i2_ref[0, 0, :] = b2 & _MASK


_dist_call = pl.pallas_call(
    _dist_body,
    grid=(_B,),
    in_specs=[pl.BlockSpec((1, _N, 8), lambda b: (b, _Z, _Z)),
              pl.BlockSpec((1, 8, _N), lambda b: (b, _Z, _Z))],
    out_specs=[pl.BlockSpec((1, 1, _N), lambda b: (b, _Z, _Z))] * 4,
    out_shape=[jax.ShapeDtypeStruct((_B, 1, _N), jnp.float32),
               jax.ShapeDtypeStruct((_B, 1, _N), jnp.int32),
               jax.ShapeDtypeStruct((_B, 1, _N), jnp.float32),
               jax.ShapeDtypeStruct((_B, 1, _N), jnp.int32)],
    scratch_shapes=[pltpu.VMEM((1, _N), jnp.float32)],
)


def _sc_body(idx_hbm, val_hbm, out_hbm,
             idx_v, val_v, zbuf, ones_v, cnt_v, accbuf, bins):
    c = lax.axis_index("c")
    s = lax.axis_index("s")
    slot = s % _PPC
    q = s // _PPC
    p = c * _PPC + slot

    def fill_ones(j, carry):
        ones_v[pl.ds(j * 16, 16)] = jnp.full((16,), 1.0, jnp.float32)
        return carry
    lax.fori_loop(jnp.int32(0), jnp.int32(128 // 16), fill_ones, jnp.int32(0))

    def fill_zero(j, carry):
        zbuf[pl.ds(j * 16, 16)] = jnp.zeros((16,), jnp.float32)
        return carry
    lax.fori_loop(jnp.int32(0), jnp.int32(1024 // 16), fill_zero, jnp.int32(0))

    pltpu.sync_copy(idx_hbm.at[p, pl.ds(q * _ROWS, _ROWS), :], idx_v)
    pltpu.sync_copy(val_hbm.at[p, pl.ds(q * _CHUNK, _CHUNK)], val_v)
    pltpu.sync_copy(zbuf, bins.at[pl.ds(s * 1024, 1024)])
    plsc.subcore_barrier()
    for k in range(_ROWS):
        pltpu.sync_copy(ones_v, bins.at[idx_v.at[jnp.int32(k)]], add=True)
    plsc.subcore_barrier()
    pltpu.sync_copy(bins, cnt_v)

    acc = jnp.zeros((16,), jnp.float32)
    for r in range(_ROWS):
        def inner(j, a):
            ii = idx_v[jnp.int32(r), pl.ds(j * 16, 16)]
            vv = val_v[pl.ds(r * 128 + j * 16, 16)]
            cc = plsc.load_gather(cnt_v, [ii])
            return a + vv / (cc + jnp.float32(_EPS))
        acc = lax.fori_loop(jnp.int32(0), jnp.int32(128 // 16), inner, acc)
    accbuf[...] = acc
    pltpu.sync_copy(accbuf, out_hbm.at[c * _NS + s])


@functools.cache
def _get_sc_call():
  return pl.kernel(
    _sc_body,
    out_type=jax.ShapeDtypeStruct((_NC * _NS, 16), jnp.float32),
    mesh=plsc.VectorSubcoreMesh(core_axis_name="c", subcore_axis_name="s"),
    compiler_params=pltpu.CompilerParams(needs_layout_passes=False),
    scratch_types=[
        pltpu.VMEM((_ROWS, 128), jnp.int32),
        pltpu.VMEM((_CHUNK,), jnp.float32),
        pltpu.VMEM((1024,), jnp.float32),
        pltpu.VMEM((128,), jnp.float32),
        pltpu.VMEM((_PPC * _N,), jnp.float32),
        pltpu.VMEM((16,), jnp.float32),
        pltpu.VMEM_SHARED((_PPC * _N,), jnp.float32),
    ],
  )


def kernel(x, gt):
    x = x.astype(jnp.float32)
    gt = gt.astype(jnp.float32)
    xt = jnp.transpose(x, (0, 2, 1))
    xx = jnp.sum(xt * xt, axis=2, keepdims=True)
    xaug = jnp.concatenate(
        [-2.0 * xt, xx, jnp.ones_like(xx), jnp.zeros((_B, _N, 3), jnp.float32)],
        axis=2)
    gg = jnp.sum(gt * gt, axis=1, keepdims=True)
    gaug = jnp.concatenate(
        [gt, jnp.ones_like(gg), gg, jnp.zeros((_B, 3, _N), jnp.float32)],
        axis=1)
    e1, i1, e2, i2 = _dist_call(xaug, gaug)
    idx_all = jnp.concatenate([i1.reshape(_B, _N), i2.reshape(_B, _N)], axis=0)
    val_all = jnp.concatenate([e1.reshape(_B, _N), e2.reshape(_B, _N)], axis=0)
    offs = (jnp.arange(_NP, dtype=jnp.int32) % _PPC * _N)[:, None]
    idx_adj = (idx_all + offs).reshape(_NP, _N // 128, 128)
    part = _get_sc_call()(idx_adj, val_all)
    total = jnp.sum(part.astype(jnp.float64))
    return 1.0 - total / (_NP * _N)


# speedup vs baseline: 1.5328x; 1.5328x over previous
import functools

import jax
import numpy as np

jax.config.update('jax_enable_x64', True)

import jax.numpy as jnp
from jax import lax
from jax.experimental import pallas as pl
from jax.experimental.pallas import tpu as pltpu
from jax.experimental.pallas import tpu_sc as plsc

_B, _D, _N = 4, 3, 4096
_C = 512
_ALPHA = 10.0
_EPS = 1e-6
_BIG = 2**30
_Z = np.int32(0)


def _i32(v):
    return v if v.dtype == jnp.int32 else v.astype(jnp.int32)

_NC, _NS = 2, 16
_PPC = 4
_NP = _B * 2
_CHUNK = _N // (_NS // _PPC)
_ROWS = _CHUNK // 128


_MASK = 0xFFF


def _dist_body(xa_ref, ga_ref, e1_ref, i1_ref, e2_ref, i2_ref, k2_scr):
    ga = ga_ref[0]
    k2_scr[...] = jnp.full((1, _N), jnp.inf, jnp.float32)

    def blk(i, carry):
        xc = xa_ref[0, pl.ds(i * _C, _C), :]
        p = jnp.zeros((_C, _N), jnp.float32)
        for dd in range(_D):
            diff = xc[:, dd:dd + 1] - ga[dd:dd + 1, :]
            p = p + diff * diff
        t = lax.bitcast_convert_type(p, jnp.int32) & jnp.int32(~_MASK)
        lane = lax.broadcasted_iota(jnp.int32, p.shape, 1)
        sub = lax.broadcasted_iota(jnp.int32, p.shape, 0) + (i * _C)
        key1 = lax.bitcast_convert_type(t | lane, jnp.float32)
        key2 = lax.bitcast_convert_type(t | sub, jnp.float32)
        m1 = jnp.min(key1, axis=1)
        b1 = lax.bitcast_convert_type(m1, jnp.int32)
        d1 = lax.bitcast_convert_type(b1 & jnp.int32(~_MASK), jnp.float32)
        e1_ref[0, 0, pl.ds(i * _C, _C)] = jnp.exp(d1 * jnp.float32(-_ALPHA))
        i1_ref[0, 0, pl.ds(i * _C, _C)] = b1 & _MASK
        m2 = jnp.min(key2, axis=0, keepdims=True)
        k2_scr[...] = jnp.minimum(k2_scr[...], m2)
        return carry

    lax.fori_loop(jnp.int32(0), jnp.int32(_N // _C), blk, jnp.int32(0))
    b2 = lax.bitcast_convert_type(k2_scr[0, :], jnp.int32)
    d2 = lax.bitcast_convert_type(b2 & jnp.int32(~_MASK), jnp.float32)
    e2_ref[0, 0, :] = jnp.exp(d2 * jnp.float32(-_ALPHA))
    i2_ref[0, 0, :] = b2 & _MASK


_dist_call = pl.pallas_call(
    _dist_body,
    grid=(_B,),
    in_specs=[pl.BlockSpec((1, _N, _D), lambda b: (b, _Z, _Z)),
              pl.BlockSpec((1, _D, _N), lambda b: (b, _Z, _Z))],
    out_specs=[pl.BlockSpec((1, 1, _N), lambda b: (b, _Z, _Z))] * 4,
    out_shape=[jax.ShapeDtypeStruct((_B, 1, _N), jnp.float32),
               jax.ShapeDtypeStruct((_B, 1, _N), jnp.int32),
               jax.ShapeDtypeStruct((_B, 1, _N), jnp.float32),
               jax.ShapeDtypeStruct((_B, 1, _N), jnp.int32)],
    scratch_shapes=[pltpu.VMEM((1, _N), jnp.float32)],
)


def _sc_body(idx_hbm, val_hbm, out_hbm,
             idx_v, val_v, zbuf, ones_v, cnt_v, accbuf, bins):
    c = lax.axis_index("c")
    s = lax.axis_index("s")
    slot = s % _PPC
    q = s // _PPC
    p = c * _PPC + slot

    def fill_ones(j, carry):
        ones_v[pl.ds(j * 16, 16)] = jnp.full((16,), 1.0, jnp.float32)
        return carry
    lax.fori_loop(jnp.int32(0), jnp.int32(128 // 16), fill_ones, jnp.int32(0))

    def fill_zero(j, carry):
        zbuf[pl.ds(j * 16, 16)] = jnp.zeros((16,), jnp.float32)
        return carry
    lax.fori_loop(jnp.int32(0), jnp.int32(1024 // 16), fill_zero, jnp.int32(0))

    pltpu.sync_copy(idx_hbm.at[p, pl.ds(q * _ROWS, _ROWS), :], idx_v)
    pltpu.sync_copy(val_hbm.at[p, pl.ds(q * _CHUNK, _CHUNK)], val_v)
    pltpu.sync_copy(zbuf, bins.at[pl.ds(s * 1024, 1024)])
    plsc.subcore_barrier()
    for k in range(_ROWS):
        pltpu.sync_copy(ones_v, bins.at[idx_v.at[jnp.int32(k)]], add=True)
    plsc.subcore_barrier()
    pltpu.sync_copy(bins, cnt_v)

    acc = jnp.zeros((16,), jnp.float32)
    for r in range(_ROWS):
        def inner(j, a):
            ii = idx_v[jnp.int32(r), pl.ds(j * 16, 16)]
            vv = val_v[pl.ds(r * 128 + j * 16, 16)]
            cc = plsc.load_gather(cnt_v, [ii])
            return a + vv / (cc + jnp.float32(_EPS))
        acc = lax.fori_loop(jnp.int32(0), jnp.int32(128 // 16), inner, acc)
    accbuf[...] = acc
    pltpu.sync_copy(accbuf, out_hbm.at[c * _NS + s])


@functools.cache
def _get_sc_call():
  return pl.kernel(
    _sc_body,
    out_type=jax.ShapeDtypeStruct((_NC * _NS, 16), jnp.float32),
    mesh=plsc.VectorSubcoreMesh(core_axis_name="c", subcore_axis_name="s"),
    compiler_params=pltpu.CompilerParams(needs_layout_passes=False),
    scratch_types=[
        pltpu.VMEM((_ROWS, 128), jnp.int32),
        pltpu.VMEM((_CHUNK,), jnp.float32),
        pltpu.VMEM((1024,), jnp.float32),
        pltpu.VMEM((128,), jnp.float32),
        pltpu.VMEM((_PPC * _N,), jnp.float32),
        pltpu.VMEM((16,), jnp.float32),
        pltpu.VMEM_SHARED((_PPC * _N,), jnp.float32),
    ],
  )


def kernel(x, gt):
    x = x.astype(jnp.float32)
    gt = gt.astype(jnp.float32)
    e1, i1, e2, i2 = _dist_call(jnp.transpose(x, (0, 2, 1)), gt)
    idx_all = jnp.concatenate([i1.reshape(_B, _N), i2.reshape(_B, _N)], axis=0)
    val_all = jnp.concatenate([e1.reshape(_B, _N), e2.reshape(_B, _N)], axis=0)
    offs = (jnp.arange(_NP, dtype=jnp.int32) % _PPC * _N)[:, None]
    idx_adj = (idx_all + offs).reshape(_NP, _N // 128, 128)
    part = _get_sc_call()(idx_adj, val_all)
    total = jnp.sum(part.astype(jnp.float64))
    return 1.0 - total / (_NP * _N)


# speedup vs baseline: 1.6814x; 1.0970x over previous
import functools

import jax
import numpy as np

jax.config.update('jax_enable_x64', True)

import jax.numpy as jnp
from jax import lax
from jax.experimental import pallas as pl
from jax.experimental.pallas import tpu as pltpu
from jax.experimental.pallas import tpu_sc as plsc

_B, _D, _N = 4, 3, 4096
_C = 512
_ALPHA = 10.0
_EPS = 1e-6
_BIG = 2**30
_Z = np.int32(0)


def _i32(v):
    return v if v.dtype == jnp.int32 else v.astype(jnp.int32)

_NC, _NS = 2, 16
_PPC = 4
_NP = _B * 2
_CHUNK = _N // (_NS // _PPC)
_ROWS = _CHUNK // 128


_MASK = 0xFFF


def _dist_body(xa_ref, ga_ref, e1_ref, i1_ref, e2_ref, i2_ref, k2_scr):
    ga = ga_ref[0]
    k2_scr[...] = jnp.full((1, _N), jnp.inf, jnp.float32)

    def blk(i, carry):
        xc = xa_ref[0, pl.ds(i * _C, _C), :]
        p = lax.dot_general(xc, ga, (((1,), (0,)), ((), ())),
                            preferred_element_type=jnp.float32)
        t = lax.bitcast_convert_type(p, jnp.int32) & jnp.int32(~_MASK)
        lane = lax.broadcasted_iota(jnp.int32, p.shape, 1)
        sub = lax.broadcasted_iota(jnp.int32, p.shape, 0) + (i * _C)
        key1 = lax.bitcast_convert_type(t | lane, jnp.float32)
        key2 = lax.bitcast_convert_type(t | sub, jnp.float32)
        m1 = jnp.min(key1, axis=1)
        b1 = lax.bitcast_convert_type(m1, jnp.int32)
        d1 = lax.bitcast_convert_type(b1 & jnp.int32(~_MASK), jnp.float32)
        e1_ref[0, 0, pl.ds(i * _C, _C)] = jnp.exp(d1 * jnp.float32(-_ALPHA))
        i1_ref[0, 0, pl.ds(i * _C, _C)] = b1 & _MASK
        m2 = jnp.min(key2, axis=0, keepdims=True)
        k2_scr[...] = jnp.minimum(k2_scr[...], m2)
        return carry

    lax.fori_loop(jnp.int32(0), jnp.int32(_N // _C), blk, jnp.int32(0))
    b2 = lax.bitcast_convert_type(k2_scr[0, :], jnp.int32)
    d2 = lax.bitcast_convert_type(b2 & jnp.int32(~_MASK), jnp.float32)
    e2_ref[0, 0, :] = jnp.exp(d2 * jnp.float32(-_ALPHA))
    i2_ref[0, 0, :] = b2 & _MASK


_dist_call = pl.pallas_call(
    _dist_body,
    grid=(_B,),
    in_specs=[pl.BlockSpec((1, _N, 16), lambda b: (b, _Z, _Z)),
              pl.BlockSpec((1, 16, _N), lambda b: (b, _Z, _Z))],
    out_specs=[pl.BlockSpec((1, 1, _N), lambda b: (b, _Z, _Z))] * 4,
    out_shape=[jax.ShapeDtypeStruct((_B, 1, _N), jnp.float32),
               jax.ShapeDtypeStruct((_B, 1, _N), jnp.int32),
               jax.ShapeDtypeStruct((_B, 1, _N), jnp.float32),
               jax.ShapeDtypeStruct((_B, 1, _N), jnp.int32)],
    scratch_shapes=[pltpu.VMEM((1, _N), jnp.float32)],
)


def _sc_body(idx_hbm, val_hbm, out_hbm,
             idx_v, val_v, zbuf, ones_v, cnt_v, accbuf, bins):
    c = lax.axis_index("c")
    s = lax.axis_index("s")
    slot = s % _PPC
    q = s // _PPC
    p = c * _PPC + slot

    def fill_ones(j, carry):
        ones_v[pl.ds(j * 16, 16)] = jnp.full((16,), 1.0, jnp.float32)
        return carry
    lax.fori_loop(jnp.int32(0), jnp.int32(128 // 16), fill_ones, jnp.int32(0))

    def fill_zero(j, carry):
        zbuf[pl.ds(j * 16, 16)] = jnp.zeros((16,), jnp.float32)
        return carry
    lax.fori_loop(jnp.int32(0), jnp.int32(1024 // 16), fill_zero, jnp.int32(0))

    pltpu.sync_copy(idx_hbm.at[p, pl.ds(q * _ROWS, _ROWS), :], idx_v)
    pltpu.sync_copy(val_hbm.at[p, pl.ds(q * _CHUNK, _CHUNK)], val_v)
    pltpu.sync_copy(zbuf, bins.at[pl.ds(s * 1024, 1024)])
    plsc.subcore_barrier()
    for k in range(_ROWS):
        pltpu.sync_copy(ones_v, bins.at[idx_v.at[jnp.int32(k)]], add=True)
    plsc.subcore_barrier()
    pltpu.sync_copy(bins, cnt_v)

    acc = jnp.zeros((16,), jnp.float32)
    for r in range(_ROWS):
        def inner(j, a):
            ii = idx_v[jnp.int32(r), pl.ds(j * 16, 16)]
            vv = val_v[pl.ds(r * 128 + j * 16, 16)]
            cc = plsc.load_gather(cnt_v, [ii])
            return a + vv / (cc + jnp.float32(_EPS))
        acc = lax.fori_loop(jnp.int32(0), jnp.int32(128 // 16), inner, acc)
    accbuf[...] = acc
    pltpu.sync_copy(accbuf, out_hbm.at[c * _NS + s])


@functools.cache
def _get_sc_call():
  return pl.kernel(
    _sc_body,
    out_type=jax.ShapeDtypeStruct((_NC * _NS, 16), jnp.float32),
    mesh=plsc.VectorSubcoreMesh(core_axis_name="c", subcore_axis_name="s"),
    compiler_params=pltpu.CompilerParams(needs_layout_passes=False),
    scratch_types=[
        pltpu.VMEM((_ROWS, 128), jnp.int32),
        pltpu.VMEM((_CHUNK,), jnp.float32),
        pltpu.VMEM((1024,), jnp.float32),
        pltpu.VMEM((128,), jnp.float32),
        pltpu.VMEM((_PPC * _N,), jnp.float32),
        pltpu.VMEM((16,), jnp.float32),
        pltpu.VMEM_SHARED((_PPC * _N,), jnp.float32),
    ],
  )


def _split(v):
    h = v.astype(jnp.bfloat16)
    return h, (v - h.astype(jnp.float32)).astype(jnp.bfloat16)


def kernel(x, gt):
    x = x.astype(jnp.float32)
    gt = gt.astype(jnp.float32)
    xt = jnp.transpose(x, (0, 2, 1))
    xx = jnp.sum(xt * xt, axis=2, keepdims=True)
    yy = jnp.sum(gt * gt, axis=1, keepdims=True)
    xh, xl = _split(-2.0 * xt)
    xxh, xxl = _split(xx)
    gh, gl = _split(gt)
    yyh, yyl = _split(yy)
    one_x = jnp.ones((_B, _N, 1), jnp.bfloat16)
    one_g = jnp.ones((_B, 1, _N), jnp.bfloat16)
    lhs = jnp.concatenate([xh, xh, xl, xl, xxh, xxl, one_x, one_x], axis=2)
    rhs = jnp.concatenate([gh, gl, gh, gl, one_g, one_g, yyh, yyl], axis=1)
    e1, i1, e2, i2 = _dist_call(lhs, rhs)
    idx_all = jnp.concatenate([i1.reshape(_B, _N), i2.reshape(_B, _N)], axis=0)
    val_all = jnp.concatenate([e1.reshape(_B, _N), e2.reshape(_B, _N)], axis=0)
    offs = (jnp.arange(_NP, dtype=jnp.int32) % _PPC * _N)[:, None]
    idx_adj = (idx_all + offs).reshape(_NP, _N // 128, 128)
    part = _get_sc_call()(idx_adj, val_all)
    total = jnp.sum(part.astype(jnp.float64))
    return 1.0 - total / (_NP * _N)
